# Initial kernel scaffold; baseline (speedup 1.0000x reference)
import jax
import jax.numpy as jnp
from jax.experimental import pallas as pl


def kernel(gene_indices, gene_values, cell_type, attention_mask, gene_table, count_W, count_b, ct_W, ct_b, g1_W, g1_b, g2_W, g2_b, b1_W, b1_b, b2_W, b2_b):
    raise NotImplementedError("write your pallas kernel here")


# speedup vs baseline: 2.3290x; 2.3290x over previous
import functools

import jax
import jax.numpy as jnp
from jax import lax
from jax.experimental import pallas as pl
from jax.experimental.pallas import tpu as pltpu
from jax.experimental.pallas import tpu_sc as plsc

B, L, V, D, C = 1024, 200, 100000, 64, 100
N = B * L
NC, NS = 2, 16
NW = NC * NS
RPT = N // NW
BPT = B // NW
CP = 128


def _film_body(ct_ref, ctW_ref, ctb_ref, g1W_ref, g1b_ref, g2W_ref, g2b_ref,
               b1W_ref, b1b_ref, b2W_ref, b2b_ref, cw_ref, cb_ref,
               gamma_ref, G_ref, A_ref):
    ce = jnp.dot(ct_ref[...], ctW_ref[...],
                 preferred_element_type=jnp.float32) + ctb_ref[...]
    h = jnp.maximum(jnp.dot(ce, g1W_ref[...],
                            preferred_element_type=jnp.float32) + g1b_ref[...], 0.0)
    gamma = jnp.dot(h, g2W_ref[...],
                    preferred_element_type=jnp.float32) + g2b_ref[...]
    hb = jnp.maximum(jnp.dot(ce, b1W_ref[...],
                             preferred_element_type=jnp.float32) + b1b_ref[...], 0.0)
    beta = jnp.dot(hb, b2W_ref[...],
                   preferred_element_type=jnp.float32) + b2b_ref[...]
    gamma_ref[...] = gamma
    G_ref[...] = gamma * cw_ref[...]
    A_ref[...] = gamma * cb_ref[...] + beta


def _film_params(ct_pad, ctWt, ctb, g1Wt, g1b, g2Wt, g2b, b1Wt, b1b, b2Wt, b2b,
                 cw, cb):
    return pl.pallas_call(
        _film_body,
        out_shape=(
            jax.ShapeDtypeStruct((B, D), jnp.float32),
            jax.ShapeDtypeStruct((B, D), jnp.float32),
            jax.ShapeDtypeStruct((B, D), jnp.float32),
        ),
    )(ct_pad, ctWt, ctb, g1Wt, g1b, g2Wt, g2b, b1Wt, b1b, b2Wt, b2b, cw, cb)


def _sc_body(table_h, idx_h, gv_h, mk_h, gam_h, G_h, A_h, out_h,
             idx_v, gv_v, mk_v, gam_v, G_v, A_v, rows_v, sem_g):
    wid = lax.axis_index("s") * NC + lax.axis_index("c")
    base = wid * RPT
    bb = wid * BPT

    pltpu.sync_copy(idx_h.at[pl.ds(base, RPT)], idx_v)
    pltpu.sync_copy(gv_h.at[pl.ds(base, RPT)], gv_v)
    pltpu.sync_copy(mk_h.at[pl.ds(base, RPT)], mk_v)
    pltpu.sync_copy(gam_h.at[pl.ds(bb, BPT)], gam_v)
    pltpu.sync_copy(G_h.at[pl.ds(bb, BPT)], G_v)
    pltpu.sync_copy(A_h.at[pl.ds(bb, BPT)], A_v)

    def per_cell(bi, _):
        off = bi * L
        c1 = pltpu.make_async_copy(
            table_h.at[idx_v.at[pl.ds(off, 104)]],
            rows_v.at[pl.ds(0, 104)], sem_g)
        c2 = pltpu.make_async_copy(
            table_h.at[idx_v.at[pl.ds(off + 104, 96)]],
            rows_v.at[pl.ds(104, 96)], sem_g)
        c1.start()
        c2.start()
        c1.wait()
        c2.wait()

        gam = [gam_v[bi, pl.ds(16 * dg, 16)] for dg in range(4)]
        Gd = [G_v[bi, pl.ds(16 * dg, 16)] for dg in range(4)]
        Ad = [A_v[bi, pl.ds(16 * dg, 16)] for dg in range(4)]

        def per_row(l, _):
            n = off + l
            iv = jnp.broadcast_to(n, (16,)).astype(jnp.int32)
            gvv = plsc.load_gather(gv_v, [iv])
            mv = plsc.load_gather(mk_v, [iv])
            for dg in range(4):
                x = rows_v[l, pl.ds(16 * dg, 16)]
                o = mv * (x * gam[dg] + (gvv * Gd[dg] + Ad[dg]))
                rows_v[l, pl.ds(16 * dg, 16)] = o
            return _

        lax.fori_loop(0, L, per_row, None)
        pltpu.sync_copy(rows_v, out_h.at[pl.ds(base + off, L)])
        return _

    lax.fori_loop(0, BPT, per_cell, None)


@functools.partial(jax.jit, static_argnames=())
def _sc_apply(table, idx_flat, gv_flat, mk_flat, gamma, G, A):
    mesh = plsc.VectorSubcoreMesh(core_axis_name="c", subcore_axis_name="s")
    return pl.kernel(
        _sc_body,
        out_type=jax.ShapeDtypeStruct((N, D), jnp.float32),
        mesh=mesh,
        compiler_params=pltpu.CompilerParams(
            needs_layout_passes=False, use_tc_tiling_on_sc=False),
        scratch_types=[
            pltpu.VMEM((RPT,), jnp.int32),
            pltpu.VMEM((RPT,), jnp.float32),
            pltpu.VMEM((RPT,), jnp.float32),
            pltpu.VMEM((BPT, D), jnp.float32),
            pltpu.VMEM((BPT, D), jnp.float32),
            pltpu.VMEM((BPT, D), jnp.float32),
            pltpu.VMEM((L, D), jnp.float32),
            pltpu.SemaphoreType.DMA,
        ],
    )(table, idx_flat, gv_flat, mk_flat, gamma, G, A)


def kernel(gene_indices, gene_values, cell_type, attention_mask, gene_table,
           count_W, count_b, ct_W, ct_b, g1_W, g1_b, g2_W, g2_b,
           b1_W, b1_b, b2_W, b2_b):
    ct_pad = jnp.zeros((B, CP), jnp.float32).at[:, :C].set(cell_type)
    ctWt = jnp.zeros((CP, D), jnp.float32).at[:C, :].set(ct_W.T)
    gamma, G, A = _film_params(
        ct_pad, ctWt, ct_b.reshape(1, D),
        g1_W.T, g1_b.reshape(1, D), g2_W.T, g2_b.reshape(1, D),
        b1_W.T, b1_b.reshape(1, D), b2_W.T, b2_b.reshape(1, D),
        count_W.reshape(1, D), count_b.reshape(1, D))
    out_flat = _sc_apply(
        gene_table,
        gene_indices.reshape(N).astype(jnp.int32),
        gene_values.reshape(N),
        attention_mask.reshape(N),
        gamma, G, A)
    return out_flat.reshape(B, L, D)
